# baseline (speedup 1.0000x reference)
import functools
import jax
import jax.numpy as jnp
from jax import lax
from jax.experimental import pallas as pl
from jax.experimental.pallas import tpu as pltpu
from jax.experimental.pallas import tpu_sc as plsc

N_ATOM = 10000
N_EDGE = 320000
D_NODE = 128
D_EDGE = 16
H = 128
N_FG = 2000
N_GRAPH = 100


_NC = 2
_NS = 16
_CHUNK = 64
_NCHUNK = N_EDGE // _CHUNK
_APAD = 10240
_ROWS_PER_TILE = _APAD // _NS
_ZROWS = 64
_NIDX = 6


def _sc_message_pass(P, EC, pidx):
  mesh = plsc.VectorSubcoreMesh(core_axis_name="c", subcore_axis_name="s")
  NW = _NC * _NS
  NI = (_NCHUNK + NW - 1) // NW + 1
  NSUP = (NI + 5) // 6

  @functools.partial(
      pl.kernel,
      mesh=mesh,
      out_type=jax.ShapeDtypeStruct((_NC, _APAD, H), jnp.float32),
      scratch_types=[
          pltpu.VMEM((_NIDX, 2, _CHUNK), jnp.int32),
          pltpu.VMEM((_CHUNK, H), jnp.float32),
          pltpu.VMEM((_CHUNK, H), jnp.float32),
          pltpu.VMEM((_CHUNK, H), jnp.float32),
          pltpu.VMEM((_CHUNK, H), jnp.float32),
          pltpu.VMEM((_CHUNK, H), jnp.float32),
          pltpu.VMEM_SHARED((_APAD, H), jnp.float32),
          pltpu.SemaphoreType.DMA,
          pltpu.SemaphoreType.DMA,
          pltpu.SemaphoreType.DMA,
          pltpu.SemaphoreType.DMA,
      ],
  )
  def k(P_hbm, EC_hbm, pidx_hbm, out_hbm, sdidx, r0, r1, r2,
        e0, e1, agg, sem_idx, sem_in0, sem_in1, sem_sc):
    c = lax.axis_index("c")
    s = lax.axis_index("s")
    wid = s * _NC + c
    rows = (r0, r1, r2)
    ecs = (e0, e1)
    sem_in = (sem_in0, sem_in1)
    zero16 = jnp.zeros((16,), jnp.float32)

    def cid(i):
      return wid + i * NW

    def issue_idx(i, slot):
      @pl.when(cid(i) < _NCHUNK)
      def _():
        pltpu.async_copy(pidx_hbm.at[cid(i)], sdidx.at[slot], sem_idx)

    def wait_idx(i, slot):
      @pl.when(cid(i) < _NCHUNK)
      def _():
        pltpu.make_async_copy(pidx_hbm.at[0], sdidx.at[slot], sem_idx).wait()

    def issue_in(i, islot, rslot, eslot):
      @pl.when(cid(i) < _NCHUNK)
      def _():
        base = cid(i) * _CHUNK
        pltpu.async_copy(P_hbm.at[sdidx.at[islot, 0]], rows[rslot],
                         sem_in[eslot])
        pltpu.async_copy(EC_hbm.at[pl.ds(base, _CHUNK)], ecs[eslot],
                         sem_in[eslot])

    def zrow(r, _):
      for j in range(H // 16):
        e0[r, pl.ds(j * 16, 16)] = zero16
      return 0
    lax.fori_loop(0, _ZROWS, zrow, 0)
    base_row = s * _ROWS_PER_TILE
    for b in range(_ROWS_PER_TILE // _ZROWS):
      pltpu.sync_copy(e0, agg.at[pl.ds(base_row + b * _ZROWS, _ZROWS)])
    plsc.subcore_barrier()

    for i in range(3):
      issue_idx(i, i)
    for i in range(2):
      wait_idx(i, i)
      issue_in(i, i, i % 3, i % 2)

    def super_body(g, _):
      for b in range(6):
        i = 6 * g + b
        rb, eb = b % 3, b % 2
        valid = cid(i) < _NCHUNK

        @pl.when(valid)
        def _():
          pltpu.make_async_copy(P_hbm.at[sdidx.at[b, 0]], rows[rb],
                                sem_in[eb]).wait()
          pltpu.make_async_copy(EC_hbm.at[pl.ds(0, _CHUNK)], ecs[eb],
                                sem_in[eb]).wait()

        @pl.when(valid)
        def _():
          def mrow(rr, _):
            for u in range(4):
              r = rr * 4 + u
              for j in range(H // 16):
                sl = pl.ds(j * 16, 16)
                rows[rb][r, sl] = jnp.maximum(
                    rows[rb][r, sl] + ecs[eb][r, sl], 0.0)
            return 0
          lax.fori_loop(0, _CHUNK // 4, mrow, 0)

        @pl.when(valid)
        def _():
          pltpu.async_copy(rows[rb], agg.at[sdidx.at[b, 1]], sem_sc,
                           add=True)

        drain_ok = (i >= 1) & (cid(i - 1) < _NCHUNK)

        @pl.when(drain_ok)
        def _():
          pltpu.make_async_copy(rows[(b - 1) % 3],
                                agg.at[sdidx.at[(b - 1) % 6, 1]],
                                sem_sc).wait()

        wait_idx(i + 2, (b + 2) % 6)
        issue_in(i + 2, (b + 2) % 6, (b + 2) % 3, b % 2)
        issue_idx(i + 3, (b + 3) % 6)
      return 0

    lax.fori_loop(0, NSUP, super_body, 0)
    plsc.subcore_barrier()

    ndrain = _ROWS_PER_TILE // _ZROWS
    for b in range(ndrain):
      if b >= 2:
        pltpu.make_async_copy(ecs[b % 2], out_hbm.at[c, pl.ds(0, _ZROWS)],
                              sem_sc).wait()
      r0_ = base_row + b * _ZROWS
      pltpu.sync_copy(agg.at[pl.ds(r0_, _ZROWS)], ecs[b % 2])
      pltpu.async_copy(ecs[b % 2], out_hbm.at[c, pl.ds(r0_, _ZROWS)], sem_sc)
    for b in range(ndrain - 2, ndrain):
      pltpu.make_async_copy(ecs[b % 2], out_hbm.at[c, pl.ds(0, _ZROWS)],
                            sem_sc).wait()

  return k(P, EC, pidx)


def _node_encode_body(feat_ref, Wne_ref, bne_ref, Wm_ref, h_ref, p_ref):
  h = jnp.maximum(
      jnp.dot(feat_ref[...], Wne_ref[...], preferred_element_type=jnp.float32)
      + bne_ref[...], 0.0)
  h_ref[...] = h
  p_ref[...] = jnp.dot(h, Wm_ref[...], preferred_element_type=jnp.float32)


def _node_encode(feat, W_ne, b_ne, Wm_top):
  blk = 1000
  grid = N_ATOM // blk
  return pl.pallas_call(
      _node_encode_body,
      grid=(grid,),
      in_specs=[
          pl.BlockSpec((blk, D_NODE), lambda i: (i, 0)),
          pl.BlockSpec((D_NODE, H), lambda i: (0, 0)),
          pl.BlockSpec((1, H), lambda i: (0, 0)),
          pl.BlockSpec((H, H), lambda i: (0, 0)),
      ],
      out_specs=[
          pl.BlockSpec((blk, H), lambda i: (i, 0)),
          pl.BlockSpec((blk, H), lambda i: (i, 0)),
      ],
      out_shape=[
          jax.ShapeDtypeStruct((N_ATOM, H), jnp.float32),
          jax.ShapeDtypeStruct((N_ATOM, H), jnp.float32),
      ],
  )(feat, W_ne, b_ne.reshape(1, H), Wm_top)


def _fg_encode_body(fgf_ref, Wse_ref, bse_ref, Wk_ref, fg_ref, k_ref):
  fg = jnp.maximum(
      jnp.dot(fgf_ref[...], Wse_ref[...], preferred_element_type=jnp.float32)
      + bse_ref[...], 0.0)
  fg_ref[...] = fg
  k_ref[...] = jnp.dot(fg, Wk_ref[...], preferred_element_type=jnp.float32)


def _fg_encode(fg_feat, W_se, b_se, W_k):
  return pl.pallas_call(
      _fg_encode_body,
      grid=(2,),
      in_specs=[
          pl.BlockSpec((N_FG // 2, D_NODE), lambda i: (i, 0)),
          pl.BlockSpec((D_NODE, H), lambda i: (0, 0)),
          pl.BlockSpec((1, H), lambda i: (0, 0)),
          pl.BlockSpec((H, H), lambda i: (0, 0)),
      ],
      out_specs=[
          pl.BlockSpec((N_FG // 2, H), lambda i: (i, 0)),
          pl.BlockSpec((N_FG // 2, H), lambda i: (i, 0)),
      ],
      out_shape=[
          jax.ShapeDtypeStruct((N_FG, H), jnp.float32),
          jax.ShapeDtypeStruct((N_FG, H), jnp.float32),
      ],
  )(fg_feat, W_se, b_se.reshape(1, H), W_k)


def _edge_encode_body(eft_ref, Wee_ref, bee_ref, Wb_ref, bm_ref, ec_ref):
  e = jnp.maximum(
      lax.dot_general(eft_ref[...], Wee_ref[...], (((0,), (0,)), ((), ())),
                      preferred_element_type=jnp.float32) + bee_ref[...], 0.0)
  ec_ref[...] = jnp.dot(e, Wb_ref[...],
                        preferred_element_type=jnp.float32) + bm_ref[...]


def _edge_encode(efeat_t, W_ee, b_ee, Wm_bot, b_msg):
  blk = 2048
  grid = (N_EDGE + blk - 1) // blk
  return pl.pallas_call(
      _edge_encode_body,
      grid=(grid,),
      in_specs=[
          pl.BlockSpec((D_EDGE, blk), lambda i: (0, i)),
          pl.BlockSpec((D_EDGE, H), lambda i: (0, 0)),
          pl.BlockSpec((1, H), lambda i: (0, 0)),
          pl.BlockSpec((H, H), lambda i: (0, 0)),
          pl.BlockSpec((1, H), lambda i: (0, 0)),
      ],
      out_specs=pl.BlockSpec((blk, H), lambda i: (i, 0)),
      out_shape=jax.ShapeDtypeStruct((N_EDGE, H), jnp.float32),
  )(efeat_t, W_ee, b_ee.reshape(1, H), Wm_bot, b_msg.reshape(1, H))


def _update_body(h_ref, a0_ref, a1_ref, Wut_ref, Wub_ref, bu_ref, Wx_ref,
                 hn_ref, x_ref):
  h = h_ref[...]
  agg = a0_ref[0] + a1_ref[0]
  hn = jnp.maximum(
      jnp.dot(h, Wut_ref[...], preferred_element_type=jnp.float32)
      + jnp.dot(agg, Wub_ref[...], preferred_element_type=jnp.float32)
      + bu_ref[...], 0.0) + h
  hn_ref[...] = hn
  x_ref[...] = jnp.dot(hn, Wx_ref[...], preferred_element_type=jnp.float32)


def _update(h, parts, Wu, bu, Wx):
  blk = 1000
  grid = N_ATOM // blk
  return pl.pallas_call(
      _update_body,
      grid=(grid,),
      in_specs=[
          pl.BlockSpec((blk, H), lambda i: (i, 0)),
          pl.BlockSpec((1, blk, H), lambda i: (0, i, 0)),
          pl.BlockSpec((1, blk, H), lambda i: (1, i, 0)),
          pl.BlockSpec((H, H), lambda i: (0, 0)),
          pl.BlockSpec((H, H), lambda i: (0, 0)),
          pl.BlockSpec((1, H), lambda i: (0, 0)),
          pl.BlockSpec((H, H), lambda i: (0, 0)),
      ],
      out_specs=[
          pl.BlockSpec((blk, H), lambda i: (i, 0)),
          pl.BlockSpec((blk, H), lambda i: (i, 0)),
      ],
      out_shape=[
          jax.ShapeDtypeStruct((N_ATOM, H), jnp.float32),
          jax.ShapeDtypeStruct((N_ATOM, H), jnp.float32),
      ],
  )(h, parts, parts, Wu[:H], Wu[H:], bu.reshape(1, H), Wx)


def _attn_body(q_ref, h_ref, agid_ref, kk_ref, fg_ref, fgid_ref, sums_ref,
               cnt_ref):
  i = pl.program_id(0)

  @pl.when(i == 0)
  def _():
    sums_ref[...] = jnp.zeros_like(sums_ref)
    cnt_ref[...] = jnp.zeros_like(cnt_ref)

  q = q_ref[...]
  agid = agid_ref[0, 0, :]
  fgid = fgid_ref[0, 0, :]
  scores = lax.dot_general(q, kk_ref[...], (((1,), (1,)), ((), ())),
                           preferred_element_type=jnp.float32)
  scores = scores * jnp.float32(1.0 / jnp.sqrt(jnp.float32(H)))
  mask = agid[:, None] == fgid[None, :]
  scores = jnp.where(mask, scores, jnp.float32(-1e9))
  smax = jnp.max(scores, axis=-1, keepdims=True)
  p = jnp.exp(scores - smax)
  attn = p / jnp.sum(p, axis=-1, keepdims=True)
  pooled = jnp.dot(attn, fg_ref[...], preferred_element_type=jnp.float32)

  onehot = (lax.broadcasted_iota(jnp.int32, (N_GRAPH, q.shape[0]), 0)
            == agid[None, :]).astype(jnp.float32)
  sums_ref[:, :H] += jnp.dot(onehot, h_ref[...],
                             preferred_element_type=jnp.float32)
  sums_ref[:, H:] += jnp.dot(onehot, pooled,
                             preferred_element_type=jnp.float32)
  cnt_ref[...] += jnp.sum(onehot, axis=1, keepdims=True) * jnp.ones(
      (1, 128), jnp.float32)


def _attn_readout(q, h, atom_gid, kk, fg, fg_gid):
  blk = 1000
  grid = N_ATOM // blk
  return pl.pallas_call(
      _attn_body,
      grid=(grid,),
      in_specs=[
          pl.BlockSpec((blk, H), lambda i: (i, 0)),
          pl.BlockSpec((blk, H), lambda i: (i, 0)),
          pl.BlockSpec((1, 1, blk), lambda i: (i, 0, 0)),
          pl.BlockSpec((N_FG, H), lambda i: (0, 0)),
          pl.BlockSpec((N_FG, H), lambda i: (0, 0)),
          pl.BlockSpec((1, 1, N_FG), lambda i: (0, 0, 0)),
      ],
      out_specs=[
          pl.BlockSpec((N_GRAPH, 2 * H), lambda i: (0, 0)),
          pl.BlockSpec((N_GRAPH, 128), lambda i: (0, 0)),
      ],
      out_shape=[
          jax.ShapeDtypeStruct((N_GRAPH, 2 * H), jnp.float32),
          jax.ShapeDtypeStruct((N_GRAPH, 128), jnp.float32),
      ],
  )(q, h, atom_gid.reshape(grid, 1, blk), kk, fg,
    fg_gid.reshape(1, 1, N_FG))


def _final_body(sums_ref, cnt_ref, Wt_ref, Wb_ref, b_ref, out_ref):
  cnt = jnp.maximum(cnt_ref[...], 1.0)
  mh = sums_ref[:, :H] / cnt
  mp = sums_ref[:, H:] / cnt
  out_ref[...] = jnp.maximum(
      jnp.dot(mh, Wt_ref[...], preferred_element_type=jnp.float32)
      + jnp.dot(mp, Wb_ref[...], preferred_element_type=jnp.float32)
      + b_ref[...], 0.0)


def _final(sums, cnt, W_fr, b_fr):
  return pl.pallas_call(
      _final_body,
      out_shape=jax.ShapeDtypeStruct((N_GRAPH, H), jnp.float32),
  )(sums, cnt, W_fr[:H], W_fr[H:], b_fr.reshape(1, H))


def kernel(feat, efeat, fg_feat, W_ne, b_ne, W_ee, b_ee, W_msg1, b_msg1,
           W_upd1, b_upd1, W_msg2, b_msg2, W_upd2, b_upd2, W_se, b_se, W_q,
           W_k, W_fr, b_fr, edge_index, atom_graph_ids, fg_graph_ids):
  pidx = (edge_index.astype(jnp.int32)
          .reshape(2, _NCHUNK, _CHUNK).transpose(1, 0, 2))
  atom_gid = atom_graph_ids.astype(jnp.int32)
  fg_gid = fg_graph_ids.astype(jnp.int32)

  efeat_t = efeat.T
  ec1 = _edge_encode(efeat_t, W_ee, b_ee, W_msg1[H:], b_msg1)
  h, p1 = _node_encode(feat, W_ne, b_ne, W_msg1[:H])
  fg, kk = _fg_encode(fg_feat, W_se, b_se, W_k)

  parts1 = _sc_message_pass(p1, ec1, pidx)
  ec2 = _edge_encode(efeat_t, W_ee, b_ee, W_msg2[H:], b_msg2)
  h, p2 = _update(h, parts1, W_upd1, b_upd1, W_msg2[:H])
  parts2 = _sc_message_pass(p2, ec2, pidx)
  h, q = _update(h, parts2, W_upd2, b_upd2, W_q)

  sums, cnt = _attn_readout(q, h, atom_gid, kk, fg, fg_gid)
  return _final(sums, cnt, W_fr, b_fr)


# speedup vs baseline: 1.0099x; 1.0099x over previous
import functools
import jax
import jax.numpy as jnp
from jax import lax
from jax.experimental import pallas as pl
from jax.experimental.pallas import tpu as pltpu
from jax.experimental.pallas import tpu_sc as plsc

N_ATOM = 10000
N_EDGE = 320000
D_NODE = 128
D_EDGE = 16
H = 128
N_FG = 2000
N_GRAPH = 100


_NC = 2
_NS = 16
_CHUNK = 64
_NCHUNK = N_EDGE // _CHUNK
_APAD = 10240
_ROWS_PER_TILE = _APAD // _NS
_ZROWS = 64
_NIDX = 6


def _pack_halves(lo, hi):
  lo16 = lax.bitcast_convert_type(lo.astype(jnp.bfloat16),
                                  jnp.uint16).astype(jnp.uint32)
  hi16 = lax.bitcast_convert_type(hi.astype(jnp.bfloat16),
                                  jnp.uint16).astype(jnp.uint32)
  return lax.bitcast_convert_type(lo16 | (hi16 << 16), jnp.int32)


def _sc_message_pass(P, EC, pidx):
  mesh = plsc.VectorSubcoreMesh(core_axis_name="c", subcore_axis_name="s")
  NW = _NC * _NS
  NI = (_NCHUNK + NW - 1) // NW + 1
  NSUP = (NI + 5) // 6

  @functools.partial(
      pl.kernel,
      mesh=mesh,
      out_type=jax.ShapeDtypeStruct((_NC, _APAD, H), jnp.float32),
      scratch_types=[
          pltpu.VMEM((_NIDX, 2, _CHUNK), jnp.int32),
          pltpu.VMEM((_CHUNK, H), jnp.float32),
          pltpu.VMEM((_CHUNK, H), jnp.float32),
          pltpu.VMEM((_CHUNK, H), jnp.float32),
          pltpu.VMEM((_CHUNK, H // 2), jnp.int32),
          pltpu.VMEM((_CHUNK, H // 2), jnp.int32),
          pltpu.VMEM_SHARED((_APAD, H), jnp.float32),
          pltpu.SemaphoreType.DMA,
          pltpu.SemaphoreType.DMA,
          pltpu.SemaphoreType.DMA,
          pltpu.SemaphoreType.DMA,
      ],
  )
  def k(P_hbm, EC_hbm, pidx_hbm, out_hbm, sdidx, r0, r1, r2,
        e0, e1, agg, sem_idx, sem_in0, sem_in1, sem_sc):
    c = lax.axis_index("c")
    s = lax.axis_index("s")
    wid = s * _NC + c
    rows = (r0, r1, r2)
    ecs = (e0, e1)
    sem_in = (sem_in0, sem_in1)
    zero16 = jnp.zeros((16,), jnp.float32)
    himask = jnp.int32(-65536)

    def cid(i):
      return wid + i * NW

    def issue_idx(i, slot):
      @pl.when(cid(i) < _NCHUNK)
      def _():
        pltpu.async_copy(pidx_hbm.at[cid(i)], sdidx.at[slot], sem_idx)

    def wait_idx(i, slot):
      @pl.when(cid(i) < _NCHUNK)
      def _():
        pltpu.make_async_copy(pidx_hbm.at[0], sdidx.at[slot], sem_idx).wait()

    def issue_in(i, islot, rslot, eslot):
      @pl.when(cid(i) < _NCHUNK)
      def _():
        base = cid(i) * _CHUNK
        pltpu.async_copy(P_hbm.at[sdidx.at[islot, 0]], rows[rslot],
                         sem_in[eslot])
        pltpu.async_copy(EC_hbm.at[pl.ds(base, _CHUNK)], ecs[eslot],
                         sem_in[eslot])

    def zrow(r, _):
      for j in range(H // 16):
        r0[r, pl.ds(j * 16, 16)] = zero16
      return 0
    lax.fori_loop(0, _ZROWS, zrow, 0)
    base_row = s * _ROWS_PER_TILE
    for b in range(_ROWS_PER_TILE // _ZROWS):
      pltpu.sync_copy(r0, agg.at[pl.ds(base_row + b * _ZROWS, _ZROWS)])
    plsc.subcore_barrier()

    for i in range(3):
      issue_idx(i, i)
    for i in range(2):
      wait_idx(i, i)
      issue_in(i, i, i % 3, i % 2)

    def super_body(g, _):
      for b in range(6):
        i = 6 * g + b
        rb, eb = b % 3, b % 2
        valid = cid(i) < _NCHUNK

        @pl.when(valid)
        def _():
          pltpu.make_async_copy(P_hbm.at[sdidx.at[b, 0]], rows[rb],
                                sem_in[eb]).wait()
          pltpu.make_async_copy(EC_hbm.at[pl.ds(0, _CHUNK)], ecs[eb],
                                sem_in[eb]).wait()

        @pl.when(valid)
        def _():
          def mrow(rr, _):
            for u in range(4):
              r = rr * 4 + u
              for j in range(H // 32):
                we = ecs[eb][r, pl.ds(j * 16, 16)]
                lo = lax.bitcast_convert_type(we << 16, jnp.float32)
                hi = lax.bitcast_convert_type(we & himask, jnp.float32)
                sl_lo = pl.ds(j * 16, 16)
                sl_hi = pl.ds(H // 2 + j * 16, 16)
                rows[rb][r, sl_lo] = jnp.maximum(rows[rb][r, sl_lo] + lo, 0.0)
                rows[rb][r, sl_hi] = jnp.maximum(rows[rb][r, sl_hi] + hi, 0.0)
            return 0
          lax.fori_loop(0, _CHUNK // 4, mrow, 0)

        @pl.when(valid)
        def _():
          pltpu.async_copy(rows[rb], agg.at[sdidx.at[b, 1]], sem_sc,
                           add=True)

        drain_ok = (i >= 1) & (cid(i - 1) < _NCHUNK)

        @pl.when(drain_ok)
        def _():
          pltpu.make_async_copy(rows[(b - 1) % 3],
                                agg.at[sdidx.at[(b - 1) % 6, 1]],
                                sem_sc).wait()

        wait_idx(i + 2, (b + 2) % 6)
        issue_in(i + 2, (b + 2) % 6, (b + 2) % 3, b % 2)
        issue_idx(i + 3, (b + 3) % 6)
      return 0

    lax.fori_loop(0, NSUP, super_body, 0)
    plsc.subcore_barrier()

    ndrain = _ROWS_PER_TILE // _ZROWS
    for b in range(ndrain):
      if b >= 2:
        pltpu.make_async_copy(rows[b % 2], out_hbm.at[c, pl.ds(0, _ZROWS)],
                              sem_sc).wait()
      r0_ = base_row + b * _ZROWS
      pltpu.sync_copy(agg.at[pl.ds(r0_, _ZROWS)], rows[b % 2])
      pltpu.async_copy(rows[b % 2], out_hbm.at[c, pl.ds(r0_, _ZROWS)], sem_sc)
    for b in range(ndrain - 2, ndrain):
      pltpu.make_async_copy(rows[b % 2], out_hbm.at[c, pl.ds(0, _ZROWS)],
                            sem_sc).wait()

  return k(P, EC, pidx)


def _node_encode_body(feat_ref, Wne_ref, bne_ref, Wm_ref, h_ref, p_ref):
  h = jnp.maximum(
      jnp.dot(feat_ref[...], Wne_ref[...], preferred_element_type=jnp.float32)
      + bne_ref[...], 0.0)
  h_ref[...] = h
  p_ref[...] = jnp.dot(h, Wm_ref[...], preferred_element_type=jnp.float32)


def _node_encode(feat, W_ne, b_ne, Wm_top):
  blk = 1000
  grid = N_ATOM // blk
  return pl.pallas_call(
      _node_encode_body,
      grid=(grid,),
      in_specs=[
          pl.BlockSpec((blk, D_NODE), lambda i: (i, 0)),
          pl.BlockSpec((D_NODE, H), lambda i: (0, 0)),
          pl.BlockSpec((1, H), lambda i: (0, 0)),
          pl.BlockSpec((H, H), lambda i: (0, 0)),
      ],
      out_specs=[
          pl.BlockSpec((blk, H), lambda i: (i, 0)),
          pl.BlockSpec((blk, H), lambda i: (i, 0)),
      ],
      out_shape=[
          jax.ShapeDtypeStruct((N_ATOM, H), jnp.float32),
          jax.ShapeDtypeStruct((N_ATOM, H), jnp.float32),
      ],
  )(feat, W_ne, b_ne.reshape(1, H), Wm_top)


def _fg_encode_body(fgf_ref, Wse_ref, bse_ref, Wk_ref, fg_ref, k_ref):
  fg = jnp.maximum(
      jnp.dot(fgf_ref[...], Wse_ref[...], preferred_element_type=jnp.float32)
      + bse_ref[...], 0.0)
  fg_ref[...] = fg
  k_ref[...] = jnp.dot(fg, Wk_ref[...], preferred_element_type=jnp.float32)


def _fg_encode(fg_feat, W_se, b_se, W_k):
  return pl.pallas_call(
      _fg_encode_body,
      grid=(2,),
      in_specs=[
          pl.BlockSpec((N_FG // 2, D_NODE), lambda i: (i, 0)),
          pl.BlockSpec((D_NODE, H), lambda i: (0, 0)),
          pl.BlockSpec((1, H), lambda i: (0, 0)),
          pl.BlockSpec((H, H), lambda i: (0, 0)),
      ],
      out_specs=[
          pl.BlockSpec((N_FG // 2, H), lambda i: (i, 0)),
          pl.BlockSpec((N_FG // 2, H), lambda i: (i, 0)),
      ],
      out_shape=[
          jax.ShapeDtypeStruct((N_FG, H), jnp.float32),
          jax.ShapeDtypeStruct((N_FG, H), jnp.float32),
      ],
  )(fg_feat, W_se, b_se.reshape(1, H), W_k)


def _edge_encode_body(eft_ref, Wee_ref, bee_ref, Wb_ref, bm_ref, ec_ref):
  e = jnp.maximum(
      lax.dot_general(eft_ref[...], Wee_ref[...], (((0,), (0,)), ((), ())),
                      preferred_element_type=jnp.float32) + bee_ref[...], 0.0)
  ec = jnp.dot(e, Wb_ref[...],
               preferred_element_type=jnp.float32) + bm_ref[...]
  ec_ref[...] = _pack_halves(ec[:, :H // 2], ec[:, H // 2:])


def _edge_encode(efeat_t, W_ee, b_ee, Wm_bot, b_msg):
  blk = 2048
  grid = (N_EDGE + blk - 1) // blk
  return pl.pallas_call(
      _edge_encode_body,
      grid=(grid,),
      in_specs=[
          pl.BlockSpec((D_EDGE, blk), lambda i: (0, i)),
          pl.BlockSpec((D_EDGE, H), lambda i: (0, 0)),
          pl.BlockSpec((1, H), lambda i: (0, 0)),
          pl.BlockSpec((H, H), lambda i: (0, 0)),
          pl.BlockSpec((1, H), lambda i: (0, 0)),
      ],
      out_specs=pl.BlockSpec((blk, H // 2), lambda i: (i, 0)),
      out_shape=jax.ShapeDtypeStruct((N_EDGE, H // 2), jnp.int32),
  )(efeat_t, W_ee, b_ee.reshape(1, H), Wm_bot, b_msg.reshape(1, H))


def _update_body(pack, h_ref, a0_ref, a1_ref, Wut_ref, Wub_ref, bu_ref,
                 Wx_ref, hn_ref, x_ref):
  h = h_ref[...]
  agg = a0_ref[0] + a1_ref[0]
  hn = jnp.maximum(
      jnp.dot(h, Wut_ref[...], preferred_element_type=jnp.float32)
      + jnp.dot(agg, Wub_ref[...], preferred_element_type=jnp.float32)
      + bu_ref[...], 0.0) + h
  hn_ref[...] = hn
  x = jnp.dot(hn, Wx_ref[...], preferred_element_type=jnp.float32)
  if pack:
    x_ref[...] = _pack_halves(x[:, :H // 2], x[:, H // 2:])
  else:
    x_ref[...] = x


def _update(h, parts, Wu, bu, Wx, pack):
  blk = 1000
  grid = N_ATOM // blk
  xw = H // 2 if pack else H
  xdt = jnp.int32 if pack else jnp.float32
  return pl.pallas_call(
      functools.partial(_update_body, pack),
      grid=(grid,),
      in_specs=[
          pl.BlockSpec((blk, H), lambda i: (i, 0)),
          pl.BlockSpec((1, blk, H), lambda i: (0, i, 0)),
          pl.BlockSpec((1, blk, H), lambda i: (1, i, 0)),
          pl.BlockSpec((H, H), lambda i: (0, 0)),
          pl.BlockSpec((H, H), lambda i: (0, 0)),
          pl.BlockSpec((1, H), lambda i: (0, 0)),
          pl.BlockSpec((H, H), lambda i: (0, 0)),
      ],
      out_specs=[
          pl.BlockSpec((blk, H), lambda i: (i, 0)),
          pl.BlockSpec((blk, xw), lambda i: (i, 0)),
      ],
      out_shape=[
          jax.ShapeDtypeStruct((N_ATOM, H), jnp.float32),
          jax.ShapeDtypeStruct((N_ATOM, xw), xdt),
      ],
  )(h, parts, parts, Wu[:H], Wu[H:], bu.reshape(1, H), Wx)


def _attn_body(q_ref, h_ref, agid_ref, kk_ref, fg_ref, fgid_ref, sums_ref,
               cnt_ref):
  i = pl.program_id(0)

  @pl.when(i == 0)
  def _():
    sums_ref[...] = jnp.zeros_like(sums_ref)
    cnt_ref[...] = jnp.zeros_like(cnt_ref)

  q = q_ref[...]
  agid = agid_ref[0, 0, :]
  fgid = fgid_ref[0, 0, :]
  scores = lax.dot_general(q, kk_ref[...], (((1,), (1,)), ((), ())),
                           preferred_element_type=jnp.float32)
  scores = scores * jnp.float32(1.0 / jnp.sqrt(jnp.float32(H)))
  mask = agid[:, None] == fgid[None, :]
  scores = jnp.where(mask, scores, jnp.float32(-1e9))
  smax = jnp.max(scores, axis=-1, keepdims=True)
  p = jnp.exp(scores - smax)
  attn = p / jnp.sum(p, axis=-1, keepdims=True)
  pooled = jnp.dot(attn, fg_ref[...], preferred_element_type=jnp.float32)

  onehot = (lax.broadcasted_iota(jnp.int32, (N_GRAPH, q.shape[0]), 0)
            == agid[None, :]).astype(jnp.float32)
  sums_ref[:, :H] += jnp.dot(onehot, h_ref[...],
                             preferred_element_type=jnp.float32)
  sums_ref[:, H:] += jnp.dot(onehot, pooled,
                             preferred_element_type=jnp.float32)
  cnt_ref[...] += jnp.sum(onehot, axis=1, keepdims=True) * jnp.ones(
      (1, 128), jnp.float32)


def _attn_readout(q, h, atom_gid, kk, fg, fg_gid):
  blk = 1000
  grid = N_ATOM // blk
  return pl.pallas_call(
      _attn_body,
      grid=(grid,),
      in_specs=[
          pl.BlockSpec((blk, H), lambda i: (i, 0)),
          pl.BlockSpec((blk, H), lambda i: (i, 0)),
          pl.BlockSpec((1, 1, blk), lambda i: (i, 0, 0)),
          pl.BlockSpec((N_FG, H), lambda i: (0, 0)),
          pl.BlockSpec((N_FG, H), lambda i: (0, 0)),
          pl.BlockSpec((1, 1, N_FG), lambda i: (0, 0, 0)),
      ],
      out_specs=[
          pl.BlockSpec((N_GRAPH, 2 * H), lambda i: (0, 0)),
          pl.BlockSpec((N_GRAPH, 128), lambda i: (0, 0)),
      ],
      out_shape=[
          jax.ShapeDtypeStruct((N_GRAPH, 2 * H), jnp.float32),
          jax.ShapeDtypeStruct((N_GRAPH, 128), jnp.float32),
      ],
  )(q, h, atom_gid.reshape(grid, 1, blk), kk, fg,
    fg_gid.reshape(1, 1, N_FG))


def _final_body(sums_ref, cnt_ref, Wt_ref, Wb_ref, b_ref, out_ref):
  cnt = jnp.maximum(cnt_ref[...], 1.0)
  mh = sums_ref[:, :H] / cnt
  mp = sums_ref[:, H:] / cnt
  out_ref[...] = jnp.maximum(
      jnp.dot(mh, Wt_ref[...], preferred_element_type=jnp.float32)
      + jnp.dot(mp, Wb_ref[...], preferred_element_type=jnp.float32)
      + b_ref[...], 0.0)


def _final(sums, cnt, W_fr, b_fr):
  return pl.pallas_call(
      _final_body,
      out_shape=jax.ShapeDtypeStruct((N_GRAPH, H), jnp.float32),
  )(sums, cnt, W_fr[:H], W_fr[H:], b_fr.reshape(1, H))


def kernel(feat, efeat, fg_feat, W_ne, b_ne, W_ee, b_ee, W_msg1, b_msg1,
           W_upd1, b_upd1, W_msg2, b_msg2, W_upd2, b_upd2, W_se, b_se, W_q,
           W_k, W_fr, b_fr, edge_index, atom_graph_ids, fg_graph_ids):
  pidx = (edge_index.astype(jnp.int32)
          .reshape(2, _NCHUNK, _CHUNK).transpose(1, 0, 2))
  atom_gid = atom_graph_ids.astype(jnp.int32)
  fg_gid = fg_graph_ids.astype(jnp.int32)

  efeat_t = efeat.T
  ec1 = _edge_encode(efeat_t, W_ee, b_ee, W_msg1[H:], b_msg1)
  h, p1 = _node_encode(feat, W_ne, b_ne, W_msg1[:H])
  fg, kk = _fg_encode(fg_feat, W_se, b_se, W_k)

  parts1 = _sc_message_pass(p1, ec1, pidx)
  ec2 = _edge_encode(efeat_t, W_ee, b_ee, W_msg2[H:], b_msg2)
  h, p2 = _update(h, parts1, W_upd1, b_upd1, W_msg2[:H], False)
  parts2 = _sc_message_pass(p2, ec2, pidx)
  h, q = _update(h, parts2, W_upd2, b_upd2, W_q, False)

  sums, cnt = _attn_readout(q, h, atom_gid, kk, fg, fg_gid)
  return _final(sums, cnt, W_fr, b_fr)
